# baseline (speedup 1.0000x reference)
import functools

import jax
import jax.numpy as jnp
from jax import lax
from jax.experimental import pallas as pl
from jax.experimental.pallas import tpu as pltpu
from jax.experimental.pallas import tpu_sc as plsc

S, D, FF, E = 2048, 1024, 3584, 8
BF = 1792
NJ = FF // BF
BM = 256
T = 4096 // BM + E
P = T * BM
META_ROWS = 32

NC, NS = 2, 16
NW = NC * NS
TOK_W = S // NW
CHT = 32


def _dispatch_body(x_hbm, p0_hbm, p1_hbm, xpad_hbm, i0_v, i1_v, buf,
                   sem0, sem1):
    wid = lax.axis_index("s") * NC + lax.axis_index("c")
    base = wid * TOK_W
    pltpu.sync_copy(p0_hbm.at[pl.ds(base, TOK_W)], i0_v)
    pltpu.sync_copy(p1_hbm.at[pl.ds(base, TOK_W)], i1_v)
    pltpu.sync_copy(x_hbm.at[pl.ds(base, TOK_W)], buf)
    c0 = pltpu.async_copy(buf, xpad_hbm.at[i0_v], sem0)
    c1 = pltpu.async_copy(buf, xpad_hbm.at[i1_v], sem1)
    c0.wait()
    c1.wait()


def _sc_dispatch(x, pos0, pos1):
    mesh = plsc.VectorSubcoreMesh(core_axis_name="c", subcore_axis_name="s")
    return pl.kernel(
        _dispatch_body,
        out_type=jax.ShapeDtypeStruct((P, D), jnp.float32),
        mesh=mesh,
        scratch_types=[
            pltpu.VMEM((TOK_W,), jnp.int32),
            pltpu.VMEM((TOK_W,), jnp.int32),
            pltpu.VMEM((TOK_W, D), jnp.float32),
            pltpu.SemaphoreType.DMA,
            pltpu.SemaphoreType.DMA,
        ],
        name="moe_sc_dispatch",
    )(x, pos0, pos1)


def _combine_body(y_hbm, p0_hbm, p1_hbm, wa_hbm, wb_hbm, out_hbm, i0_v,
                  i1_v, wa_v, wb_v, bufa, bufb, sema, semb):
    wid = lax.axis_index("s") * NC + lax.axis_index("c")
    base = wid * TOK_W
    for c in range(TOK_W // CHT):
        off = base + c * CHT
        pltpu.sync_copy(p0_hbm.at[pl.ds(off, CHT)], i0_v)
        pltpu.sync_copy(p1_hbm.at[pl.ds(off, CHT)], i1_v)
        pltpu.sync_copy(wa_hbm.at[pl.ds(off, CHT)], wa_v)
        pltpu.sync_copy(wb_hbm.at[pl.ds(off, CHT)], wb_v)
        ca = pltpu.async_copy(y_hbm.at[i0_v], bufa, sema)
        cb = pltpu.async_copy(y_hbm.at[i1_v], bufb, semb)
        ca.wait()
        cb.wait()

        def _row_fma(r, carry):
            wa = wa_v[r, :]
            wb = wb_v[r, :]
            for k in range(D // 16):
                sl = pl.ds(k * 16, 16)
                bufa[r, sl] = bufa[r, sl] * wa + bufb[r, sl] * wb
            return carry

        lax.fori_loop(0, CHT, _row_fma, 0)
        pltpu.sync_copy(bufa, out_hbm.at[pl.ds(off, CHT)])


def _sc_combine(y_pad, pos0, pos1, w0b, w1b):
    mesh = plsc.VectorSubcoreMesh(core_axis_name="c", subcore_axis_name="s")
    return pl.kernel(
        _combine_body,
        out_type=jax.ShapeDtypeStruct((S, D), jnp.float32),
        mesh=mesh,
        scratch_types=[
            pltpu.VMEM((CHT,), jnp.int32),
            pltpu.VMEM((CHT,), jnp.int32),
            pltpu.VMEM((CHT, 16), jnp.float32),
            pltpu.VMEM((CHT, 16), jnp.float32),
            pltpu.VMEM((CHT, D), jnp.float32),
            pltpu.VMEM((CHT, D), jnp.float32),
            pltpu.SemaphoreType.DMA,
            pltpu.SemaphoreType.DMA,
        ],
        name="moe_sc_combine",
    )(y_pad, pos0, pos1, w0b, w1b)


def _router_body(x_ref, g_ref, logits_ref, pos0_ref, pos1_ref, p1_ref,
                 p2_ref, meta_ref):
    x = x_ref[...]
    g = g_ref[...]
    logits = jnp.dot(x, g, preferred_element_type=jnp.float32)
    logits_ref[...] = logits
    lane = jax.lax.broadcasted_iota(jnp.int32, logits.shape, 1)
    m1 = jnp.max(logits, axis=1, keepdims=True)
    i1 = jnp.min(jnp.where(logits == m1, lane, E), axis=1, keepdims=True)
    mask1 = lane == i1
    l2 = jnp.max(jnp.where(mask1, -jnp.inf, logits), axis=1, keepdims=True)
    i2 = jnp.min(
        jnp.where((logits == l2) & (~mask1), lane, E), axis=1, keepdims=True
    )
    mask2 = lane == i2
    p1_ref[...] = jax.nn.sigmoid(m1 - l2)
    p2_ref[...] = jax.nn.sigmoid(l2 - m1)

    oh = mask1.astype(jnp.int32) + mask2.astype(jnp.int32)
    inc = oh
    sh = 1
    while sh < S:
        shifted = jnp.concatenate(
            [jnp.zeros((sh, E), jnp.int32), inc[: S - sh, :]], axis=0
        )
        inc = inc + shifted
        sh *= 2
    exc = inc - oh
    g_cnt = inc[S - 1 : S, :]
    tiles_per = (g_cnt + (BM - 1)) // BM
    tinc = tiles_per
    sh = 1
    while sh < E:
        tinc = tinc + jnp.concatenate(
            [jnp.zeros((1, sh), jnp.int32), tinc[:, : E - sh]], axis=1
        )
        sh *= 2
    tstart = tinc - tiles_per
    base = tstart * BM
    rank1 = jnp.sum(jnp.where(mask1, exc, 0), axis=1, keepdims=True)
    rank2 = jnp.sum(jnp.where(mask2, exc, 0), axis=1, keepdims=True)
    base1 = jnp.sum(jnp.where(mask1, base, 0), axis=1, keepdims=True)
    base2 = jnp.sum(jnp.where(mask2, base, 0), axis=1, keepdims=True)
    pos0_ref[...] = base1 + rank1
    pos1_ref[...] = base2 + rank2
    tq = jax.lax.broadcasted_iota(jnp.int32, (META_ROWS, E), 0)
    te = jnp.sum((tq >= tstart).astype(jnp.int32), axis=1, keepdims=True) - 1
    n_tiles = jnp.sum(tiles_per, axis=1, keepdims=True)
    row = jax.lax.broadcasted_iota(jnp.int32, (META_ROWS, 1), 0)
    meta_ref[...] = jnp.where(row == T, jnp.broadcast_to(n_tiles,
                                                         (META_ROWS, 1)), te)


def _mlp_body(te_ref, nt_ref, xp_ref, w1_ref, w3_ref, w2_ref, y_ref):
    t = pl.program_id(0)
    j = pl.program_id(1)

    @pl.when(t < nt_ref[0])
    def _compute():
        x = xp_ref[...]
        h1 = jnp.dot(x, w1_ref[0], preferred_element_type=jnp.float32)
        h1 = h1 * jax.nn.sigmoid(h1)
        h3 = jnp.dot(x, w3_ref[0], preferred_element_type=jnp.float32)
        part = jnp.dot(h1 * h3, w2_ref[0], preferred_element_type=jnp.float32)

        @pl.when(j == 0)
        def _init():
            y_ref[...] = part

        @pl.when(j > 0)
        def _acc():
            y_ref[...] += part


@jax.jit
def kernel(hidden_states, gate_w, w1, w3, w2):
    x = hidden_states.reshape(S, D)

    logits, pos0, pos1, p1, p2, meta = pl.pallas_call(
        _router_body,
        grid=(1,),
        in_specs=[
            pl.BlockSpec((S, D), lambda i: (0, 0)),
            pl.BlockSpec((D, E), lambda i: (0, 0)),
        ],
        out_specs=[
            pl.BlockSpec((S, E), lambda i: (0, 0)),
            pl.BlockSpec((S, 1), lambda i: (0, 0)),
            pl.BlockSpec((S, 1), lambda i: (0, 0)),
            pl.BlockSpec((S, 1), lambda i: (0, 0)),
            pl.BlockSpec((S, 1), lambda i: (0, 0)),
            pl.BlockSpec((META_ROWS, 1), lambda i: (0, 0)),
        ],
        out_shape=[
            jax.ShapeDtypeStruct((S, E), jnp.float32),
            jax.ShapeDtypeStruct((S, 1), jnp.int32),
            jax.ShapeDtypeStruct((S, 1), jnp.int32),
            jax.ShapeDtypeStruct((S, 1), jnp.float32),
            jax.ShapeDtypeStruct((S, 1), jnp.float32),
            jax.ShapeDtypeStruct((META_ROWS, 1), jnp.int32),
        ],
    )(x, gate_w)

    tile_expert = meta[:T, 0]
    n_tiles = meta[T, :]
    pos0_f = pos0.reshape(S)
    pos1_f = pos1.reshape(S)

    x_pad = _sc_dispatch(x, pos0_f, pos1_f)

    y_pad = pl.pallas_call(
        _mlp_body,
        grid_spec=pltpu.PrefetchScalarGridSpec(
            num_scalar_prefetch=2,
            grid=(T, NJ),
            in_specs=[
                pl.BlockSpec((BM, D), lambda t, j, te, nt: (t, 0)),
                pl.BlockSpec((1, D, BF), lambda t, j, te, nt: (te[t], 0, j)),
                pl.BlockSpec((1, D, BF), lambda t, j, te, nt: (te[t], 0, j)),
                pl.BlockSpec((1, BF, D), lambda t, j, te, nt: (te[t], j, 0)),
            ],
            out_specs=pl.BlockSpec((BM, D), lambda t, j, te, nt: (t, 0)),
        ),
        out_shape=jax.ShapeDtypeStruct((P, D), jnp.float32),
        compiler_params=pltpu.CompilerParams(
            dimension_semantics=("arbitrary", "arbitrary"),
        ),
    )(tile_expert, n_tiles, x_pad, w1, w3, w2)

    w0b = jnp.broadcast_to(p1, (S, 16))
    w1b = jnp.broadcast_to(p2, (S, 16))
    out = _sc_combine(y_pad, pos0_f, pos1_f, w0b, w1b)

    return out.reshape(1, S, D), logits.reshape(1, S, E)


# speedup vs baseline: 1.3196x; 1.3196x over previous
import functools

import jax
import jax.numpy as jnp
from jax import lax
from jax.experimental import pallas as pl
from jax.experimental.pallas import tpu as pltpu
from jax.experimental.pallas import tpu_sc as plsc

S, D, FF, E = 2048, 1024, 3584, 8
BF = 1792
NJ = FF // BF
BM = 576
T = 4096 // BM + E
P = T * BM
META_ROWS = 32

NC, NS = 2, 16
NW = NC * NS
TOK_W = S // NW
CHT = 32


def _dispatch_body(x_hbm, p0_hbm, p1_hbm, xpad_hbm, i0_v, i1_v, buf,
                   sem0, sem1):
    wid = lax.axis_index("s") * NC + lax.axis_index("c")
    base = wid * TOK_W
    pltpu.sync_copy(p0_hbm.at[pl.ds(base, TOK_W)], i0_v)
    pltpu.sync_copy(p1_hbm.at[pl.ds(base, TOK_W)], i1_v)
    pltpu.sync_copy(x_hbm.at[pl.ds(base, TOK_W)], buf)
    c0 = pltpu.async_copy(buf, xpad_hbm.at[i0_v], sem0)
    c1 = pltpu.async_copy(buf, xpad_hbm.at[i1_v], sem1)
    c0.wait()
    c1.wait()


def _sc_dispatch(x, pos0, pos1):
    mesh = plsc.VectorSubcoreMesh(core_axis_name="c", subcore_axis_name="s")
    return pl.kernel(
        _dispatch_body,
        out_type=jax.ShapeDtypeStruct((P, D), jnp.float32),
        mesh=mesh,
        scratch_types=[
            pltpu.VMEM((TOK_W,), jnp.int32),
            pltpu.VMEM((TOK_W,), jnp.int32),
            pltpu.VMEM((TOK_W, D), jnp.float32),
            pltpu.SemaphoreType.DMA,
            pltpu.SemaphoreType.DMA,
        ],
        name="moe_sc_dispatch",
    )(x, pos0, pos1)


def _combine_body(y_hbm, p0_hbm, p1_hbm, wa_hbm, wb_hbm, out_hbm, i0_v,
                  i1_v, wa_v, wb_v, bufa, bufb, sema, semb):
    wid = lax.axis_index("s") * NC + lax.axis_index("c")
    base = wid * TOK_W
    for c in range(TOK_W // CHT):
        off = base + c * CHT
        pltpu.sync_copy(p0_hbm.at[pl.ds(off, CHT)], i0_v)
        pltpu.sync_copy(p1_hbm.at[pl.ds(off, CHT)], i1_v)
        pltpu.sync_copy(wa_hbm.at[pl.ds(off, CHT)], wa_v)
        pltpu.sync_copy(wb_hbm.at[pl.ds(off, CHT)], wb_v)
        ca = pltpu.async_copy(y_hbm.at[i0_v], bufa, sema)
        cb = pltpu.async_copy(y_hbm.at[i1_v], bufb, semb)
        ca.wait()
        cb.wait()

        def _row_fma(r, carry):
            wa = wa_v[r, :]
            wb = wb_v[r, :]
            for k in range(D // 16):
                sl = pl.ds(k * 16, 16)
                bufa[r, sl] = bufa[r, sl] * wa + bufb[r, sl] * wb
            return carry

        lax.fori_loop(0, CHT, _row_fma, 0)
        pltpu.sync_copy(bufa, out_hbm.at[pl.ds(off, CHT)])


def _sc_combine(y_pad, pos0, pos1, w0b, w1b):
    mesh = plsc.VectorSubcoreMesh(core_axis_name="c", subcore_axis_name="s")
    return pl.kernel(
        _combine_body,
        out_type=jax.ShapeDtypeStruct((S, D), jnp.float32),
        mesh=mesh,
        scratch_types=[
            pltpu.VMEM((CHT,), jnp.int32),
            pltpu.VMEM((CHT,), jnp.int32),
            pltpu.VMEM((CHT, 16), jnp.float32),
            pltpu.VMEM((CHT, 16), jnp.float32),
            pltpu.VMEM((CHT, D), jnp.float32),
            pltpu.VMEM((CHT, D), jnp.float32),
            pltpu.SemaphoreType.DMA,
            pltpu.SemaphoreType.DMA,
        ],
        name="moe_sc_combine",
    )(y_pad, pos0, pos1, w0b, w1b)


def _router_body(x_ref, g_ref, logits_ref, pos0_ref, pos1_ref, p1_ref,
                 p2_ref, meta_ref):
    x = x_ref[...]
    g = g_ref[...]
    logits = jnp.dot(x, g, preferred_element_type=jnp.float32)
    logits_ref[...] = logits
    lane = jax.lax.broadcasted_iota(jnp.int32, logits.shape, 1)
    m1 = jnp.max(logits, axis=1, keepdims=True)
    i1 = jnp.min(jnp.where(logits == m1, lane, E), axis=1, keepdims=True)
    mask1 = lane == i1
    l2 = jnp.max(jnp.where(mask1, -jnp.inf, logits), axis=1, keepdims=True)
    i2 = jnp.min(
        jnp.where((logits == l2) & (~mask1), lane, E), axis=1, keepdims=True
    )
    mask2 = lane == i2
    p1_ref[...] = jax.nn.sigmoid(m1 - l2)
    p2_ref[...] = jax.nn.sigmoid(l2 - m1)

    oh = mask1.astype(jnp.int32) + mask2.astype(jnp.int32)
    inc = oh
    sh = 1
    while sh < S:
        shifted = jnp.concatenate(
            [jnp.zeros((sh, E), jnp.int32), inc[: S - sh, :]], axis=0
        )
        inc = inc + shifted
        sh *= 2
    exc = inc - oh
    g_cnt = inc[S - 1 : S, :]
    tiles_per = (g_cnt + (BM - 1)) // BM
    tinc = tiles_per
    sh = 1
    while sh < E:
        tinc = tinc + jnp.concatenate(
            [jnp.zeros((1, sh), jnp.int32), tinc[:, : E - sh]], axis=1
        )
        sh *= 2
    tstart = tinc - tiles_per
    base = tstart * BM
    rank1 = jnp.sum(jnp.where(mask1, exc, 0), axis=1, keepdims=True)
    rank2 = jnp.sum(jnp.where(mask2, exc, 0), axis=1, keepdims=True)
    base1 = jnp.sum(jnp.where(mask1, base, 0), axis=1, keepdims=True)
    base2 = jnp.sum(jnp.where(mask2, base, 0), axis=1, keepdims=True)
    pos0_ref[...] = base1 + rank1
    pos1_ref[...] = base2 + rank2
    tq = jax.lax.broadcasted_iota(jnp.int32, (META_ROWS, E), 0)
    te = jnp.sum((tq >= tstart).astype(jnp.int32), axis=1, keepdims=True) - 1
    n_tiles = jnp.sum(tiles_per, axis=1, keepdims=True)
    row = jax.lax.broadcasted_iota(jnp.int32, (META_ROWS, 1), 0)
    meta_ref[...] = jnp.where(row == T, jnp.broadcast_to(n_tiles,
                                                         (META_ROWS, 1)), te)


def _mlp_body(te_ref, nt_ref, xp_ref, w1_ref, w3_ref, w2_ref, y_ref):
    t = pl.program_id(0)
    j = pl.program_id(1)

    @pl.when(t < nt_ref[0])
    def _compute():
        x = xp_ref[...]
        h1 = jnp.dot(x, w1_ref[0], preferred_element_type=jnp.float32)
        h1 = h1 * jax.nn.sigmoid(h1)
        h3 = jnp.dot(x, w3_ref[0], preferred_element_type=jnp.float32)
        part = jnp.dot(h1 * h3, w2_ref[0], preferred_element_type=jnp.float32)

        @pl.when(j == 0)
        def _init():
            y_ref[...] = part

        @pl.when(j > 0)
        def _acc():
            y_ref[...] += part


@jax.jit
def kernel(hidden_states, gate_w, w1, w3, w2):
    x = hidden_states.reshape(S, D)

    logits, pos0, pos1, p1, p2, meta = pl.pallas_call(
        _router_body,
        grid=(1,),
        in_specs=[
            pl.BlockSpec((S, D), lambda i: (0, 0)),
            pl.BlockSpec((D, E), lambda i: (0, 0)),
        ],
        out_specs=[
            pl.BlockSpec((S, E), lambda i: (0, 0)),
            pl.BlockSpec((S, 1), lambda i: (0, 0)),
            pl.BlockSpec((S, 1), lambda i: (0, 0)),
            pl.BlockSpec((S, 1), lambda i: (0, 0)),
            pl.BlockSpec((S, 1), lambda i: (0, 0)),
            pl.BlockSpec((META_ROWS, 1), lambda i: (0, 0)),
        ],
        out_shape=[
            jax.ShapeDtypeStruct((S, E), jnp.float32),
            jax.ShapeDtypeStruct((S, 1), jnp.int32),
            jax.ShapeDtypeStruct((S, 1), jnp.int32),
            jax.ShapeDtypeStruct((S, 1), jnp.float32),
            jax.ShapeDtypeStruct((S, 1), jnp.float32),
            jax.ShapeDtypeStruct((META_ROWS, 1), jnp.int32),
        ],
    )(x, gate_w)

    tile_expert = meta[:T, 0]
    n_tiles = meta[T, :]
    pos0_f = pos0.reshape(S)
    pos1_f = pos1.reshape(S)

    x_pad = _sc_dispatch(x, pos0_f, pos1_f)

    y_pad = pl.pallas_call(
        _mlp_body,
        grid_spec=pltpu.PrefetchScalarGridSpec(
            num_scalar_prefetch=2,
            grid=(T, NJ),
            in_specs=[
                pl.BlockSpec((BM, D), lambda t, j, te, nt: (t, 0)),
                pl.BlockSpec((1, D, BF), lambda t, j, te, nt: (te[t], 0, j)),
                pl.BlockSpec((1, D, BF), lambda t, j, te, nt: (te[t], 0, j)),
                pl.BlockSpec((1, BF, D), lambda t, j, te, nt: (te[t], j, 0)),
            ],
            out_specs=pl.BlockSpec((BM, D), lambda t, j, te, nt: (t, 0)),
        ),
        out_shape=jax.ShapeDtypeStruct((P, D), jnp.float32),
        compiler_params=pltpu.CompilerParams(
            dimension_semantics=("arbitrary", "arbitrary"),
        ),
    )(tile_expert, n_tiles, x_pad, w1, w3, w2)

    w0b = jnp.broadcast_to(p1, (S, 16))
    w1b = jnp.broadcast_to(p2, (S, 16))
    out = _sc_combine(y_pad, pos0_f, pos1_f, w0b, w1b)

    return out.reshape(1, S, D), logits.reshape(1, S, E)


# speedup vs baseline: 1.3741x; 1.0413x over previous
import functools

import jax
import jax.numpy as jnp
from jax import lax
from jax.experimental import pallas as pl
from jax.experimental.pallas import tpu as pltpu
from jax.experimental.pallas import tpu_sc as plsc

S, D, FF, E = 2048, 1024, 3584, 8
BF = 1792
NJ = FF // BF
BM = 576
T = 4096 // BM + E
P = T * BM
META_ROWS = 32

NC, NS = 2, 16
NW = NC * NS
TOK_W = S // NW
CHT = 32


def _dispatch_body(x_hbm, p0_hbm, p1_hbm, xpad_hbm, i0_v, i1_v, buf,
                   sem0, sem1):
    wid = lax.axis_index("s") * NC + lax.axis_index("c")
    base = wid * TOK_W
    pltpu.sync_copy(p0_hbm.at[pl.ds(base, TOK_W)], i0_v)
    pltpu.sync_copy(p1_hbm.at[pl.ds(base, TOK_W)], i1_v)
    pltpu.sync_copy(x_hbm.at[pl.ds(base, TOK_W)], buf)
    c0 = pltpu.async_copy(buf, xpad_hbm.at[i0_v], sem0)
    c1 = pltpu.async_copy(buf, xpad_hbm.at[i1_v], sem1)
    c0.wait()
    c1.wait()


def _sc_dispatch(x, pos0, pos1):
    mesh = plsc.VectorSubcoreMesh(core_axis_name="c", subcore_axis_name="s")
    return pl.kernel(
        _dispatch_body,
        out_type=jax.ShapeDtypeStruct((P, D), jnp.float32),
        mesh=mesh,
        scratch_types=[
            pltpu.VMEM((TOK_W,), jnp.int32),
            pltpu.VMEM((TOK_W,), jnp.int32),
            pltpu.VMEM((TOK_W, D), jnp.float32),
            pltpu.SemaphoreType.DMA,
            pltpu.SemaphoreType.DMA,
        ],
        name="moe_sc_dispatch",
    )(x, pos0, pos1)


def _combine_body(y_hbm, p0_hbm, p1_hbm, wa_hbm, wb_hbm, out_hbm, i0_v,
                  i1_v, wa_v, wb_v, bufa, bufb, sema, semb):
    wid = lax.axis_index("s") * NC + lax.axis_index("c")
    base = wid * TOK_W
    for c in range(TOK_W // CHT):
        off = base + c * CHT
        pltpu.sync_copy(p0_hbm.at[pl.ds(off, CHT)], i0_v)
        pltpu.sync_copy(p1_hbm.at[pl.ds(off, CHT)], i1_v)
        pltpu.sync_copy(wa_hbm.at[pl.ds(off, CHT)], wa_v)
        pltpu.sync_copy(wb_hbm.at[pl.ds(off, CHT)], wb_v)
        ca = pltpu.async_copy(y_hbm.at[i0_v], bufa, sema)
        cb = pltpu.async_copy(y_hbm.at[i1_v], bufb, semb)
        ca.wait()
        cb.wait()

        def _row_fma(r, carry):
            wa = wa_v[r, :]
            wb = wb_v[r, :]
            for k in range(D // 16):
                sl = pl.ds(k * 16, 16)
                bufa[r, sl] = bufa[r, sl] * wa + bufb[r, sl] * wb
            return carry

        lax.fori_loop(0, CHT, _row_fma, 0)
        pltpu.sync_copy(bufa, out_hbm.at[pl.ds(off, CHT)])


def _sc_combine(y_pad, pos0, pos1, w0b, w1b):
    mesh = plsc.VectorSubcoreMesh(core_axis_name="c", subcore_axis_name="s")
    return pl.kernel(
        _combine_body,
        out_type=jax.ShapeDtypeStruct((S, D), jnp.float32),
        mesh=mesh,
        scratch_types=[
            pltpu.VMEM((CHT,), jnp.int32),
            pltpu.VMEM((CHT,), jnp.int32),
            pltpu.VMEM((CHT, 16), jnp.float32),
            pltpu.VMEM((CHT, 16), jnp.float32),
            pltpu.VMEM((CHT, D), jnp.float32),
            pltpu.VMEM((CHT, D), jnp.float32),
            pltpu.SemaphoreType.DMA,
            pltpu.SemaphoreType.DMA,
        ],
        name="moe_sc_combine",
    )(y_pad, pos0, pos1, w0b, w1b)


def _router_body(x_ref, g_ref, logits_ref, pos0_ref, pos1_ref, p1_ref,
                 p2_ref, meta_ref):
    x = x_ref[...]
    g = g_ref[...]
    logits = jnp.dot(x, g, preferred_element_type=jnp.float32)
    logits_ref[...] = logits
    lane = jax.lax.broadcasted_iota(jnp.int32, logits.shape, 1)
    m1 = jnp.max(logits, axis=1, keepdims=True)
    i1 = jnp.min(jnp.where(logits == m1, lane, E), axis=1, keepdims=True)
    mask1 = lane == i1
    l2 = jnp.max(jnp.where(mask1, -jnp.inf, logits), axis=1, keepdims=True)
    i2 = jnp.min(
        jnp.where((logits == l2) & (~mask1), lane, E), axis=1, keepdims=True
    )
    mask2 = lane == i2
    p1_ref[...] = jax.nn.sigmoid(m1 - l2)
    p2_ref[...] = jax.nn.sigmoid(l2 - m1)

    oh = mask1.astype(jnp.int32) + mask2.astype(jnp.int32)
    inc = oh
    sh = 1
    while sh < S:
        shifted = jnp.concatenate(
            [jnp.zeros((sh, E), jnp.int32), inc[: S - sh, :]], axis=0
        )
        inc = inc + shifted
        sh *= 2
    exc = inc - oh
    g_cnt = inc[S - 1 : S, :]
    tiles_per = (g_cnt + (BM - 1)) // BM
    tinc = tiles_per
    sh = 1
    while sh < E:
        tinc = tinc + jnp.concatenate(
            [jnp.zeros((1, sh), jnp.int32), tinc[:, : E - sh]], axis=1
        )
        sh *= 2
    tstart = tinc - tiles_per
    base = tstart * BM
    rank1 = jnp.sum(jnp.where(mask1, exc, 0), axis=1, keepdims=True)
    rank2 = jnp.sum(jnp.where(mask2, exc, 0), axis=1, keepdims=True)
    base1 = jnp.sum(jnp.where(mask1, base, 0), axis=1, keepdims=True)
    base2 = jnp.sum(jnp.where(mask2, base, 0), axis=1, keepdims=True)
    pos0_ref[...] = base1 + rank1
    pos1_ref[...] = base2 + rank2
    tq = jax.lax.broadcasted_iota(jnp.int32, (META_ROWS, E), 0)
    te = jnp.sum((tq >= tstart).astype(jnp.int32), axis=1, keepdims=True) - 1
    n_tiles = jnp.sum(tiles_per, axis=1, keepdims=True)
    row = jax.lax.broadcasted_iota(jnp.int32, (META_ROWS, 1), 0)
    meta_ref[...] = jnp.where(row == T, jnp.broadcast_to(n_tiles,
                                                         (META_ROWS, 1)), te)


def _mlp_body(te_ref, nt_ref, xp_ref, w1_ref, w3_ref, w2_ref, y_ref):
    t = pl.program_id(0)
    j = pl.program_id(1)

    @pl.when(t < nt_ref[0])
    def _compute():
        x = xp_ref[...]
        h1 = jnp.dot(x, w1_ref[0], preferred_element_type=jnp.float32)
        h1 = h1 * jax.nn.sigmoid(h1)
        h3 = jnp.dot(x, w3_ref[0], preferred_element_type=jnp.float32)
        part = jnp.dot(h1 * h3, w2_ref[0], preferred_element_type=jnp.float32)

        @pl.when(j == 0)
        def _init():
            y_ref[...] = part

        @pl.when(j > 0)
        def _acc():
            y_ref[...] += part


@jax.jit
def kernel(hidden_states, gate_w, w1, w3, w2):
    x = hidden_states.reshape(S, D)

    logits, pos0, pos1, p1, p2, meta = pl.pallas_call(
        _router_body,
        grid=(1,),
        in_specs=[
            pl.BlockSpec((S, D), lambda i: (0, 0)),
            pl.BlockSpec((D, E), lambda i: (0, 0)),
        ],
        out_specs=[
            pl.BlockSpec((S, E), lambda i: (0, 0)),
            pl.BlockSpec((S, 1), lambda i: (0, 0)),
            pl.BlockSpec((S, 1), lambda i: (0, 0)),
            pl.BlockSpec((S, 1), lambda i: (0, 0)),
            pl.BlockSpec((S, 1), lambda i: (0, 0)),
            pl.BlockSpec((META_ROWS, 1), lambda i: (0, 0)),
        ],
        out_shape=[
            jax.ShapeDtypeStruct((S, E), jnp.float32),
            jax.ShapeDtypeStruct((S, 1), jnp.int32),
            jax.ShapeDtypeStruct((S, 1), jnp.int32),
            jax.ShapeDtypeStruct((S, 1), jnp.float32),
            jax.ShapeDtypeStruct((S, 1), jnp.float32),
            jax.ShapeDtypeStruct((META_ROWS, 1), jnp.int32),
        ],
    )(x, gate_w)

    tile_expert = meta[:T, 0]
    n_tiles = meta[T, :]
    pos0_f = pos0.reshape(S)
    pos1_f = pos1.reshape(S)

    x_pad = _sc_dispatch(x, pos0_f, pos1_f)

    y_pad = pl.pallas_call(
        _mlp_body,
        grid_spec=pltpu.PrefetchScalarGridSpec(
            num_scalar_prefetch=2,
            grid=(T, NJ),
            in_specs=[
                pl.BlockSpec(
                    (BM, D),
                    lambda t, j, te, nt: (jnp.minimum(t, nt[0] - 1), 0),
                ),
                pl.BlockSpec(
                    (1, D, BF),
                    lambda t, j, te, nt: (te[jnp.minimum(t, nt[0] - 1)], 0, j),
                ),
                pl.BlockSpec(
                    (1, D, BF),
                    lambda t, j, te, nt: (te[jnp.minimum(t, nt[0] - 1)], 0, j),
                ),
                pl.BlockSpec(
                    (1, BF, D),
                    lambda t, j, te, nt: (te[jnp.minimum(t, nt[0] - 1)], j, 0),
                ),
            ],
            out_specs=pl.BlockSpec(
                (BM, D), lambda t, j, te, nt: (jnp.minimum(t, nt[0] - 1), 0)
            ),
        ),
        out_shape=jax.ShapeDtypeStruct((P, D), jnp.float32),
        compiler_params=pltpu.CompilerParams(
            dimension_semantics=("arbitrary", "arbitrary"),
        ),
    )(tile_expert, n_tiles, x_pad, w1, w3, w2)

    w0b = jnp.broadcast_to(p1, (S, 16))
    w1b = jnp.broadcast_to(p2, (S, 16))
    out = _sc_combine(y_pad, pos0_f, pos1_f, w0b, w1b)

    return out.reshape(1, S, D), logits.reshape(1, S, E)
